# baseline (speedup 1.0000x reference)
import functools

import jax
import jax.numpy as jnp
from jax import lax
from jax.experimental import pallas as pl
from jax.experimental.pallas import tpu as pltpu
from jax.experimental.pallas import tpu_sc as plsc

NUM_CORES = 2
NUM_SUBCORES = 16
LANES = 16
CHUNK = 128


def _sc_spmm(u_tiles, dst, w_exp, src_exp, n):
    e = w_exp.shape[0]
    e_half = e // NUM_CORES
    chunks = e_half // CHUNK

    mesh = plsc.VectorSubcoreMesh(
        core_axis_name="c", subcore_axis_name="s",
        num_cores=NUM_CORES, num_subcores=NUM_SUBCORES)

    @functools.partial(
        pl.kernel,
        out_type=jax.ShapeDtypeStruct(
            (NUM_CORES, NUM_SUBCORES, n, LANES), jnp.float32),
        mesh=mesh,
        compiler_params=pltpu.CompilerParams(
            use_tc_tiling_on_sc=False, needs_layout_passes=False),
        scratch_types=[
            pltpu.VMEM((CHUNK,), jnp.int32),
            pltpu.VMEM((CHUNK,), jnp.int32),
            pltpu.VMEM((CHUNK, LANES), jnp.float32),
            pltpu.VMEM((CHUNK, LANES), jnp.int32),
            pltpu.VMEM((CHUNK, LANES), jnp.float32),
            pltpu.VMEM((n, LANES), jnp.float32),
            pltpu.SemaphoreType.DMA,
        ],
    )
    def sc_kernel(u_hbm, dst_hbm, wexp_hbm, sexp_hbm, out_hbm,
                  dst_v, idx_v, wexp_v, sexp_v, rows_v, acc_v, sem):
        cid = lax.axis_index("c")
        sid = lax.axis_index("s")
        row0 = sid * n
        ebase = cid * e_half

        zero = jnp.zeros((LANES,), jnp.float32)

        def zero_row(r, _):
            acc_v[r] = zero
            return 0

        lax.fori_loop(0, n, zero_row, 0)

        lane_iota = lax.iota(jnp.int32, LANES)

        def do_chunk(c, _):
            off = ebase + c * CHUNK
            pltpu.sync_copy(dst_hbm.at[pl.ds(off, CHUNK)], dst_v)
            pltpu.sync_copy(wexp_hbm.at[pl.ds(off, CHUNK)], wexp_v)
            pltpu.sync_copy(sexp_hbm.at[pl.ds(off, CHUNK)], sexp_v)

            def mkidx(g, _):
                sl = pl.ds(g * LANES, LANES)
                idx_v[sl] = dst_v[sl] + row0
                return 0

            lax.fori_loop(0, CHUNK // LANES, mkidx, 0)
            pltpu.async_copy(u_hbm.at[idx_v], rows_v, sem).wait()

            def acc_edge(r, _):
                plsc.addupdate_scatter(
                    acc_v, [sexp_v[r], lane_iota], rows_v[r] * wexp_v[r])
                return 0

            lax.fori_loop(0, CHUNK, acc_edge, 0, unroll=4)
            return 0

        lax.fori_loop(0, chunks, do_chunk, 0)
        pltpu.sync_copy(acc_v, out_hbm.at[cid, sid])

    return sc_kernel(u_tiles, dst, w_exp, src_exp)


TC_BLK = 512


def _tc_finish_body(acc_ref, w_ref, b_ref, out_ref, ssq_ref):
    i = pl.program_id(0)

    @pl.when(i == 0)
    def _():
        ssq_ref[0] = 0.0

    a = acc_ref[0] + acc_ref[1]
    half = NUM_SUBCORES // 2
    left = jnp.concatenate([a[t] for t in range(half)], axis=1)
    right = jnp.concatenate([a[t] for t in range(half, NUM_SUBCORES)], axis=1)
    s = jnp.dot(left, w_ref[...], preferred_element_type=jnp.float32) + b_ref[...]
    t = jnp.dot(right, w_ref[...], preferred_element_type=jnp.float32) + b_ref[...]
    diff = jnp.maximum(s, 0.0) - jnp.maximum(t, 0.0)
    ssq_ref[0] += jnp.sum(diff * diff)

    @pl.when(i == pl.num_programs(0) - 1)
    def _():
        out_ref[...] = jnp.exp(-jnp.sqrt(ssq_ref[0])).reshape(1, 1)


def kernel(x, edge_index, edge_weight, feat_a, W, b):
    n, d_in = x.shape
    e = edge_weight.shape[0]
    d_hid = W.shape[1]
    u = jnp.concatenate([x, feat_a], axis=1)
    u_tiles = u.reshape(n, NUM_SUBCORES, LANES).transpose(1, 0, 2)
    u_tiles = u_tiles.reshape(NUM_SUBCORES * n, LANES)
    w_exp = jnp.broadcast_to(edge_weight[:, None], (e, LANES))
    src_exp = jnp.broadcast_to(edge_index[0][:, None], (e, LANES))
    acc = _sc_spmm(u_tiles, edge_index[1], w_exp, src_exp, n)
    out = pl.pallas_call(
        _tc_finish_body,
        grid=(n // TC_BLK,),
        in_specs=[
            pl.BlockSpec((NUM_CORES, NUM_SUBCORES, TC_BLK, LANES),
                         lambda i: (0, 0, i, 0)),
            pl.BlockSpec((d_in, d_hid), lambda i: (0, 0)),
            pl.BlockSpec((1, d_hid), lambda i: (0, 0)),
        ],
        out_specs=pl.BlockSpec((1, 1), lambda i: (0, 0)),
        out_shape=jax.ShapeDtypeStruct((1, 1), jnp.float32),
        scratch_shapes=[pltpu.SMEM((1,), jnp.float32)],
    )(acc, W, b.reshape(1, -1))
    return out[0, 0]


# speedup vs baseline: 3.0653x; 3.0653x over previous
import functools

import jax
import jax.numpy as jnp
from jax import lax
from jax.experimental import pallas as pl
from jax.experimental.pallas import tpu as pltpu
from jax.experimental.pallas import tpu_sc as plsc

NUM_CORES = 2
NUM_SUBCORES = 16
LANES = 16
CHUNK = 128
SUPER = 512
CPS = SUPER // CHUNK


def _sc_spmm(u_tiles, src, dst, w, n):
    e = w.shape[0]
    e_half = e // NUM_CORES
    n_super = e_half // SUPER

    mesh = plsc.VectorSubcoreMesh(
        core_axis_name="c", subcore_axis_name="s",
        num_cores=NUM_CORES, num_subcores=NUM_SUBCORES)

    @functools.partial(
        pl.kernel,
        out_type=jax.ShapeDtypeStruct(
            (NUM_CORES, NUM_SUBCORES, n, LANES), jnp.float32),
        mesh=mesh,
        compiler_params=pltpu.CompilerParams(
            use_tc_tiling_on_sc=False, needs_layout_passes=False),
        scratch_types=[
            pltpu.VMEM((2, SUPER), jnp.int32),
            pltpu.VMEM((2, SUPER), jnp.int32),
            pltpu.VMEM((2, SUPER), jnp.float32),
            pltpu.VMEM((2, CHUNK), jnp.int32),
            pltpu.VMEM((2, CHUNK, LANES), jnp.float32),
            pltpu.VMEM((n, LANES), jnp.float32),
            pltpu.SemaphoreType.DMA,
            pltpu.SemaphoreType.DMA,
            pltpu.SemaphoreType.DMA,
            pltpu.SemaphoreType.DMA,
        ],
    )
    def sc_kernel(u_hbm, src_hbm, dst_hbm, w_hbm, out_hbm,
                  dst2, src2, w2, idx2, rows2, acc_v,
                  msem0, msem1, gsem0, gsem1):
        msem = (msem0, msem1)
        gsem = (gsem0, gsem1)
        cid = lax.axis_index("c")
        sid = lax.axis_index("s")
        row0 = sid * n
        ebase = cid * e_half

        zero = jnp.zeros((LANES,), jnp.float32)

        def zero_row(r, _):
            acc_v[r] = zero
            return 0

        lax.fori_loop(0, n, zero_row, 0)

        lane_iota = lax.iota(jnp.int32, LANES)

        def meta_descs(t, mp):
            off = ebase + lax.rem(t, n_super) * SUPER
            sl = pl.ds(off, SUPER)
            return ((dst_hbm.at[sl], dst2.at[mp]),
                    (src_hbm.at[sl], src2.at[mp]),
                    (w_hbm.at[sl], w2.at[mp]))

        def issue_meta(t, mp):
            for s_ref, d_ref in meta_descs(t, mp):
                pltpu.async_copy(s_ref, d_ref, msem[mp])

        def wait_meta(t, mp):
            for s_ref, d_ref in meta_descs(t, mp):
                pltpu.make_async_copy(s_ref, d_ref, msem[mp]).wait()

        def mkidx(mp, koff, gp):
            def body(g, _):
                sl = pl.ds(g * LANES, LANES)
                idx2[gp, sl] = dst2[mp, pl.ds(koff * CHUNK + g * LANES, LANES)] + row0
                return 0

            lax.fori_loop(0, CHUNK // LANES, body, 0)

        def issue_gather(gp):
            pltpu.async_copy(u_hbm.at[idx2.at[gp]], rows2.at[gp], gsem[gp])

        def wait_gather(gp):
            pltpu.make_async_copy(
                u_hbm.at[idx2.at[gp]], rows2.at[gp], gsem[gp]).wait()

        def process(mp, k, gp):
            def group(g, _):
                base = k * CHUNK + g * LANES
                wg = w2[mp, pl.ds(base, LANES)]
                sg = src2[mp, pl.ds(base, LANES)]
                rbase = g * LANES
                for j in range(LANES):
                    sel = jnp.full((LANES,), j, jnp.int32)
                    wj = jnp.take_along_axis(
                        wg, sel, axis=0, mode="promise_in_bounds")
                    sj = jnp.take_along_axis(
                        sg, sel, axis=0, mode="promise_in_bounds")
                    plsc.addupdate_scatter(
                        acc_v, [sj, lane_iota], rows2[gp, rbase + j] * wj)
                return 0

            lax.fori_loop(0, CHUNK // LANES, group, 0)

        issue_meta(0, 0)
        wait_meta(0, 0)
        mkidx(0, 0, 0)
        issue_gather(0)
        issue_meta(1, 1)

        def outer(jj, _):
            for phase in range(2):
                s = 2 * jj + phase
                for k in range(CPS):
                    gp = k % 2
                    gq = (k + 1) % 2
                    if k == CPS - 1:
                        wait_meta(s + 1, (phase + 1) % 2)
                        mkidx((phase + 1) % 2, 0, gq)
                    else:
                        mkidx(phase, k + 1, gq)
                    issue_gather(gq)
                    wait_gather(gp)
                    process(phase, k, gp)
                    if k == CPS - 1:
                        issue_meta(s + 2, phase)
            return 0

        lax.fori_loop(0, n_super // 2, outer, 0)

        wait_gather(0)
        wait_meta(n_super + 1, 1)

        pltpu.sync_copy(acc_v, out_hbm.at[cid, sid])

    return sc_kernel(u_tiles, src, dst, w)


TC_BLK = 512


def _tc_finish_body(acc_ref, w_ref, b_ref, out_ref, ssq_ref):
    i = pl.program_id(0)

    @pl.when(i == 0)
    def _():
        ssq_ref[0] = 0.0

    a = acc_ref[0] + acc_ref[1]
    half = NUM_SUBCORES // 2
    left = jnp.concatenate([a[t] for t in range(half)], axis=1)
    right = jnp.concatenate([a[t] for t in range(half, NUM_SUBCORES)], axis=1)
    s = jnp.dot(left, w_ref[...], preferred_element_type=jnp.float32) + b_ref[...]
    t = jnp.dot(right, w_ref[...], preferred_element_type=jnp.float32) + b_ref[...]
    diff = jnp.maximum(s, 0.0) - jnp.maximum(t, 0.0)
    ssq_ref[0] += jnp.sum(diff * diff)

    @pl.when(i == pl.num_programs(0) - 1)
    def _():
        out_ref[...] = jnp.exp(-jnp.sqrt(ssq_ref[0])).reshape(1, 1)


def kernel(x, edge_index, edge_weight, feat_a, W, b):
    n, d_in = x.shape
    d_hid = W.shape[1]
    u = jnp.concatenate([x, feat_a], axis=1)
    u_tiles = u.reshape(n, NUM_SUBCORES, LANES).transpose(1, 0, 2)
    u_tiles = u_tiles.reshape(NUM_SUBCORES * n, LANES)
    acc = _sc_spmm(u_tiles, edge_index[0], edge_index[1], edge_weight, n)
    out = pl.pallas_call(
        _tc_finish_body,
        grid=(n // TC_BLK,),
        in_specs=[
            pl.BlockSpec((NUM_CORES, NUM_SUBCORES, TC_BLK, LANES),
                         lambda i: (0, 0, i, 0)),
            pl.BlockSpec((d_in, d_hid), lambda i: (0, 0)),
            pl.BlockSpec((1, d_hid), lambda i: (0, 0)),
        ],
        out_specs=pl.BlockSpec((1, 1), lambda i: (0, 0)),
        out_shape=jax.ShapeDtypeStruct((1, 1), jnp.float32),
        scratch_shapes=[pltpu.SMEM((1,), jnp.float32)],
    )(acc, W, b.reshape(1, -1))
    return out[0, 0]


# speedup vs baseline: 3.3568x; 1.0951x over previous
import functools

import jax
import jax.numpy as jnp
from jax import lax
from jax.experimental import pallas as pl
from jax.experimental.pallas import tpu as pltpu
from jax.experimental.pallas import tpu_sc as plsc

NUM_CORES = 2
NUM_SUBCORES = 16
LANES = 16
CHUNK = 128
SUPER = 512
CPS = SUPER // CHUNK


def _sc_spmm(u_tiles, src, dst, w, n):
    e = w.shape[0]
    e_half = e // NUM_CORES
    n_super = e_half // SUPER

    mesh = plsc.VectorSubcoreMesh(
        core_axis_name="c", subcore_axis_name="s",
        num_cores=NUM_CORES, num_subcores=NUM_SUBCORES)

    @functools.partial(
        pl.kernel,
        out_type=jax.ShapeDtypeStruct(
            (NUM_CORES, NUM_SUBCORES, n * LANES), jnp.float32),
        mesh=mesh,
        compiler_params=pltpu.CompilerParams(
            use_tc_tiling_on_sc=False, needs_layout_passes=False),
        scratch_types=[
            pltpu.VMEM((2, SUPER), jnp.int32),
            pltpu.VMEM((2, SUPER), jnp.int32),
            pltpu.VMEM((2, SUPER), jnp.float32),
            pltpu.VMEM((2, CHUNK), jnp.int32),
            pltpu.VMEM((2, CHUNK, LANES), jnp.float32),
            pltpu.VMEM((n * LANES,), jnp.float32),
            pltpu.SemaphoreType.DMA,
            pltpu.SemaphoreType.DMA,
            pltpu.SemaphoreType.DMA,
            pltpu.SemaphoreType.DMA,
        ],
    )
    def sc_kernel(u_hbm, src_hbm, dst_hbm, w_hbm, out_hbm,
                  dst2, src2, w2, idx2, rows2, acc_v,
                  msem0, msem1, gsem0, gsem1):
        msem = (msem0, msem1)
        gsem = (gsem0, gsem1)
        cid = lax.axis_index("c")
        sid = lax.axis_index("s")
        row0 = sid * n
        ebase = cid * e_half

        zero = jnp.zeros((LANES,), jnp.float32)

        @plsc.parallel_loop(0, n * LANES, step=LANES, unroll=8)
        def zero_row(r):
            acc_v[pl.ds(r, LANES)] = zero

        lane_iota = lax.iota(jnp.int32, LANES)

        def meta_descs(t, mp):
            off = ebase + lax.rem(t, n_super) * SUPER
            sl = pl.ds(off, SUPER)
            return ((dst_hbm.at[sl], dst2.at[mp]),
                    (src_hbm.at[sl], src2.at[mp]),
                    (w_hbm.at[sl], w2.at[mp]))

        def issue_meta(t, mp):
            for s_ref, d_ref in meta_descs(t, mp):
                pltpu.async_copy(s_ref, d_ref, msem[mp])

        def wait_meta(t, mp):
            for s_ref, d_ref in meta_descs(t, mp):
                pltpu.make_async_copy(s_ref, d_ref, msem[mp]).wait()

        def mkidx(mp, koff, gp):
            @plsc.parallel_loop(0, CHUNK, step=LANES, unroll=2)
            def body(i):
                idx2[gp, pl.ds(i, LANES)] = (
                    dst2[mp, pl.ds(koff * CHUNK + i, LANES)] + row0)

        def issue_gather(gp):
            pltpu.async_copy(u_hbm.at[idx2.at[gp]], rows2.at[gp], gsem[gp])

        def wait_gather(gp):
            pltpu.make_async_copy(
                u_hbm.at[idx2.at[gp]], rows2.at[gp], gsem[gp]).wait()

        def process(mp, k, gp):
            @plsc.parallel_loop(0, CHUNK // LANES, unroll=2)
            def group(g):
                base = k * CHUNK + g * LANES
                wg = w2[mp, pl.ds(base, LANES)]
                sg16 = src2[mp, pl.ds(base, LANES)] << 4
                rbase = g * LANES
                for j in range(LANES):
                    sel = jnp.full((LANES,), j, jnp.int32)
                    wj = jnp.take_along_axis(
                        wg, sel, axis=0, mode="promise_in_bounds")
                    aj = jnp.take_along_axis(
                        sg16, sel, axis=0, mode="promise_in_bounds") | lane_iota
                    plsc.addupdate_scatter(
                        acc_v, [aj], rows2[gp, rbase + j] * wj)

        issue_meta(0, 0)
        wait_meta(0, 0)
        mkidx(0, 0, 0)
        issue_gather(0)
        issue_meta(1, 1)

        def outer(jj, _):
            for phase in range(2):
                s = 2 * jj + phase
                for k in range(CPS):
                    gp = k % 2
                    gq = (k + 1) % 2
                    if k == CPS - 1:
                        wait_meta(s + 1, (phase + 1) % 2)
                        mkidx((phase + 1) % 2, 0, gq)
                    else:
                        mkidx(phase, k + 1, gq)
                    issue_gather(gq)
                    wait_gather(gp)
                    process(phase, k, gp)
                    if k == CPS - 1:
                        issue_meta(s + 2, phase)
            return 0

        lax.fori_loop(0, n_super // 2, outer, 0)

        wait_gather(0)
        wait_meta(n_super + 1, 1)

        pltpu.sync_copy(acc_v, out_hbm.at[cid, sid])

    return sc_kernel(u_tiles, src, dst, w)


TC_BLK = 512


def _tc_finish_body(acc_ref, w_ref, b_ref, out_ref, ssq_ref):
    i = pl.program_id(0)

    @pl.when(i == 0)
    def _():
        ssq_ref[0] = 0.0

    a = acc_ref[0] + acc_ref[1]
    half = NUM_SUBCORES // 2
    left = jnp.concatenate([a[t] for t in range(half)], axis=1)
    right = jnp.concatenate([a[t] for t in range(half, NUM_SUBCORES)], axis=1)
    s = jnp.dot(left, w_ref[...], preferred_element_type=jnp.float32) + b_ref[...]
    t = jnp.dot(right, w_ref[...], preferred_element_type=jnp.float32) + b_ref[...]
    diff = jnp.maximum(s, 0.0) - jnp.maximum(t, 0.0)
    ssq_ref[0] += jnp.sum(diff * diff)

    @pl.when(i == pl.num_programs(0) - 1)
    def _():
        out_ref[...] = jnp.exp(-jnp.sqrt(ssq_ref[0])).reshape(1, 1)


def kernel(x, edge_index, edge_weight, feat_a, W, b):
    n, d_in = x.shape
    d_hid = W.shape[1]
    u = jnp.concatenate([x, feat_a], axis=1)
    u_tiles = u.reshape(n, NUM_SUBCORES, LANES).transpose(1, 0, 2)
    u_tiles = u_tiles.reshape(NUM_SUBCORES * n, LANES)
    acc = _sc_spmm(u_tiles, edge_index[0], edge_index[1], edge_weight, n)
    acc = acc.reshape(NUM_CORES, NUM_SUBCORES, n, LANES)
    out = pl.pallas_call(
        _tc_finish_body,
        grid=(n // TC_BLK,),
        in_specs=[
            pl.BlockSpec((NUM_CORES, NUM_SUBCORES, TC_BLK, LANES),
                         lambda i: (0, 0, i, 0)),
            pl.BlockSpec((d_in, d_hid), lambda i: (0, 0)),
            pl.BlockSpec((1, d_hid), lambda i: (0, 0)),
        ],
        out_specs=pl.BlockSpec((1, 1), lambda i: (0, 0)),
        out_shape=jax.ShapeDtypeStruct((1, 1), jnp.float32),
        scratch_shapes=[pltpu.SMEM((1,), jnp.float32)],
    )(acc, W, b.reshape(1, -1))
    return out[0, 0]


# speedup vs baseline: 3.8122x; 1.1357x over previous
import functools

import jax
import jax.numpy as jnp
from jax import lax
from jax.experimental import pallas as pl
from jax.experimental.pallas import tpu as pltpu
from jax.experimental.pallas import tpu_sc as plsc

NUM_CORES = 2
NUM_SUBCORES = 16
LANES = 16
CHUNK = 128
SUPER = 512
CPS = SUPER // CHUNK


def _sc_spmm(u_tiles, src, dst, w, n):
    e = w.shape[0]
    e_half = e // NUM_CORES
    n_super = e_half // SUPER

    mesh = plsc.VectorSubcoreMesh(
        core_axis_name="c", subcore_axis_name="s",
        num_cores=NUM_CORES, num_subcores=NUM_SUBCORES)

    @functools.partial(
        pl.kernel,
        out_type=jax.ShapeDtypeStruct(
            (NUM_CORES, NUM_SUBCORES, n * LANES), jnp.float32),
        mesh=mesh,
        compiler_params=pltpu.CompilerParams(
            use_tc_tiling_on_sc=False, needs_layout_passes=False),
        scratch_types=[
            pltpu.VMEM((2, SUPER), jnp.int32),
            pltpu.VMEM((2, SUPER), jnp.int32),
            pltpu.VMEM((2, SUPER), jnp.float32),
            pltpu.VMEM((2, CHUNK), jnp.int32),
            pltpu.VMEM((2, CHUNK, LANES), jnp.float32),
            pltpu.VMEM((n * LANES,), jnp.float32),
            pltpu.SemaphoreType.DMA,
            pltpu.SemaphoreType.DMA,
            pltpu.SemaphoreType.DMA,
            pltpu.SemaphoreType.DMA,
        ],
    )
    def sc_kernel(u_hbm, src_hbm, dst_hbm, w_hbm, out_hbm,
                  dst2, src2, w2, idx2, rows2, acc_v,
                  msem0, msem1, gsem0, gsem1):
        msem = (msem0, msem1)
        gsem = (gsem0, gsem1)
        cid = lax.axis_index("c")
        sid = lax.axis_index("s")
        ebase = cid * e_half

        zero = jnp.zeros((LANES,), jnp.float32)

        @plsc.parallel_loop(0, n * LANES, step=LANES, unroll=8)
        def zero_row(r):
            acc_v[pl.ds(r, LANES)] = zero

        lane_iota = lax.iota(jnp.int32, LANES)

        def meta_descs(t, mp):
            off = ebase + lax.rem(t, n_super) * SUPER
            sl = pl.ds(off, SUPER)
            return ((dst_hbm.at[sl], dst2.at[mp]),
                    (src_hbm.at[sl], src2.at[mp]),
                    (w_hbm.at[sl], w2.at[mp]))

        def issue_meta(t, mp):
            for s_ref, d_ref in meta_descs(t, mp):
                pltpu.async_copy(s_ref, d_ref, msem[mp])

        def wait_meta(t, mp):
            for s_ref, d_ref in meta_descs(t, mp):
                pltpu.make_async_copy(s_ref, d_ref, msem[mp]).wait()

        def mkidx(mp, koff, gp):
            @plsc.parallel_loop(0, CHUNK, step=LANES, unroll=2)
            def body(i):
                idx2[gp, pl.ds(i, LANES)] = (
                    (dst2[mp, pl.ds(koff * CHUNK + i, LANES)] << 4) + sid)

        def issue_gather(gp):
            pltpu.async_copy(u_hbm.at[idx2.at[gp]], rows2.at[gp], gsem[gp])

        def wait_gather(gp):
            pltpu.make_async_copy(
                u_hbm.at[idx2.at[gp]], rows2.at[gp], gsem[gp]).wait()

        def process(mp, k, gp):
            @plsc.parallel_loop(0, CHUNK // LANES, unroll=2)
            def group(g):
                base = k * CHUNK + g * LANES
                wg = w2[mp, pl.ds(base, LANES)]
                sg16 = src2[mp, pl.ds(base, LANES)] << 4
                rbase = g * LANES
                for j in range(LANES):
                    sel = jnp.full((LANES,), j, jnp.int32)
                    wj = jnp.take_along_axis(
                        wg, sel, axis=0, mode="promise_in_bounds")
                    aj = jnp.take_along_axis(
                        sg16, sel, axis=0, mode="promise_in_bounds") | lane_iota
                    plsc.addupdate_scatter(
                        acc_v, [aj], rows2[gp, rbase + j] * wj)

        issue_meta(0, 0)
        wait_meta(0, 0)
        mkidx(0, 0, 0)
        issue_gather(0)
        issue_meta(1, 1)

        def outer(jj, _):
            for phase in range(2):
                s = 2 * jj + phase
                for k in range(CPS):
                    gp = k % 2
                    gq = (k + 1) % 2
                    if k == CPS - 1:
                        wait_meta(s + 1, (phase + 1) % 2)
                        mkidx((phase + 1) % 2, 0, gq)
                    else:
                        mkidx(phase, k + 1, gq)
                    issue_gather(gq)
                    wait_gather(gp)
                    process(phase, k, gp)
                    if k == CPS - 1:
                        issue_meta(s + 2, phase)
            return 0

        lax.fori_loop(0, n_super // 2, outer, 0)

        wait_gather(0)
        wait_meta(n_super + 1, 1)

        pltpu.sync_copy(acc_v, out_hbm.at[cid, sid])

    return sc_kernel(u_tiles, src, dst, w)


TC_BLK = 512


def _tc_finish_body(acc_ref, w_ref, b_ref, out_ref, ssq_ref):
    i = pl.program_id(0)

    @pl.when(i == 0)
    def _():
        ssq_ref[0] = 0.0

    a = acc_ref[0] + acc_ref[1]
    half = NUM_SUBCORES // 2
    left = jnp.concatenate([a[t] for t in range(half)], axis=1)
    right = jnp.concatenate([a[t] for t in range(half, NUM_SUBCORES)], axis=1)
    s = jnp.dot(left, w_ref[...], preferred_element_type=jnp.float32) + b_ref[...]
    t = jnp.dot(right, w_ref[...], preferred_element_type=jnp.float32) + b_ref[...]
    diff = jnp.maximum(s, 0.0) - jnp.maximum(t, 0.0)
    ssq_ref[0] += jnp.sum(diff * diff)

    @pl.when(i == pl.num_programs(0) - 1)
    def _():
        out_ref[...] = jnp.exp(-jnp.sqrt(ssq_ref[0])).reshape(1, 1)


def kernel(x, edge_index, edge_weight, feat_a, W, b):
    n, d_in = x.shape
    d_hid = W.shape[1]
    u = jnp.concatenate([x, feat_a], axis=1)
    u_tiles = u.reshape(n * NUM_SUBCORES, LANES)
    acc = _sc_spmm(u_tiles, edge_index[0], edge_index[1], edge_weight, n)
    acc = acc.reshape(NUM_CORES, NUM_SUBCORES, n, LANES)
    out = pl.pallas_call(
        _tc_finish_body,
        grid=(n // TC_BLK,),
        in_specs=[
            pl.BlockSpec((NUM_CORES, NUM_SUBCORES, TC_BLK, LANES),
                         lambda i: (0, 0, i, 0)),
            pl.BlockSpec((d_in, d_hid), lambda i: (0, 0)),
            pl.BlockSpec((1, d_hid), lambda i: (0, 0)),
        ],
        out_specs=pl.BlockSpec((1, 1), lambda i: (0, 0)),
        out_shape=jax.ShapeDtypeStruct((1, 1), jnp.float32),
        scratch_shapes=[pltpu.SMEM((1,), jnp.float32)],
    )(acc, W, b.reshape(1, -1))
    return out[0, 0]


# speedup vs baseline: 4.0352x; 1.0585x over previous
import functools

import jax
import jax.numpy as jnp
from jax import lax
from jax.experimental import pallas as pl
from jax.experimental.pallas import tpu as pltpu
from jax.experimental.pallas import tpu_sc as plsc

NUM_CORES = 2
NUM_SUBCORES = 16
LANES = 16
CHUNK = 128
SUPER = 512
CPS = SUPER // CHUNK


def _sc_spmm(u_tiles, src, dst, w, n):
    e = w.shape[0]
    e_half = e // NUM_CORES
    n_super = e_half // SUPER

    mesh = plsc.VectorSubcoreMesh(
        core_axis_name="c", subcore_axis_name="s",
        num_cores=NUM_CORES, num_subcores=NUM_SUBCORES)

    @functools.partial(
        pl.kernel,
        out_type=jax.ShapeDtypeStruct(
            (NUM_CORES, NUM_SUBCORES, n * LANES), jnp.float32),
        mesh=mesh,
        compiler_params=pltpu.CompilerParams(
            use_tc_tiling_on_sc=False, needs_layout_passes=False),
        scratch_types=[
            pltpu.VMEM((2, SUPER), jnp.int32),
            pltpu.VMEM((2, SUPER), jnp.int32),
            pltpu.VMEM((2, SUPER), jnp.float32),
            pltpu.VMEM((4, CHUNK), jnp.int32),
            pltpu.VMEM((4, CHUNK, LANES), jnp.float32),
            pltpu.VMEM((n * LANES,), jnp.float32),
            pltpu.SemaphoreType.DMA,
            pltpu.SemaphoreType.DMA,
            pltpu.SemaphoreType.DMA,
            pltpu.SemaphoreType.DMA,
            pltpu.SemaphoreType.DMA,
            pltpu.SemaphoreType.DMA,
        ],
    )
    def sc_kernel(u_hbm, src_hbm, dst_hbm, w_hbm, out_hbm,
                  dst2, src2, w2, idx2, rows2, acc_v,
                  msem0, msem1, gsem0, gsem1, gsem2, gsem3):
        msem = (msem0, msem1)
        gsem = (gsem0, gsem1, gsem2, gsem3)
        cid = lax.axis_index("c")
        sid = lax.axis_index("s")
        ebase = cid * e_half

        zero = jnp.zeros((LANES,), jnp.float32)

        @plsc.parallel_loop(0, n * LANES, step=LANES, unroll=8)
        def zero_row(r):
            acc_v[pl.ds(r, LANES)] = zero

        lane_iota = lax.iota(jnp.int32, LANES)

        def meta_descs(t, mp):
            off = ebase + lax.rem(t, n_super) * SUPER
            sl = pl.ds(off, SUPER)
            return ((dst_hbm.at[sl], dst2.at[mp]),
                    (src_hbm.at[sl], src2.at[mp]),
                    (w_hbm.at[sl], w2.at[mp]))

        def issue_meta(t, mp):
            for s_ref, d_ref in meta_descs(t, mp):
                pltpu.async_copy(s_ref, d_ref, msem[mp])

        def wait_meta(t, mp):
            for s_ref, d_ref in meta_descs(t, mp):
                pltpu.make_async_copy(s_ref, d_ref, msem[mp]).wait()

        def mkidx(mp, koff, gp):
            @plsc.parallel_loop(0, CHUNK, step=LANES, unroll=2)
            def body(i):
                idx2[gp, pl.ds(i, LANES)] = (
                    (dst2[mp, pl.ds(koff * CHUNK + i, LANES)] << 4) + sid)

        def issue_gather(gp):
            pltpu.async_copy(u_hbm.at[idx2.at[gp]], rows2.at[gp], gsem[gp])

        def wait_gather(gp):
            pltpu.make_async_copy(
                u_hbm.at[idx2.at[gp]], rows2.at[gp], gsem[gp]).wait()

        def process(mp, k, gp):
            @plsc.parallel_loop(0, CHUNK // LANES, unroll=2)
            def group(g):
                base = k * CHUNK + g * LANES
                wg = w2[mp, pl.ds(base, LANES)]
                sg16 = src2[mp, pl.ds(base, LANES)] << 4
                rbase = g * LANES
                for j in range(LANES):
                    sel = jnp.full((LANES,), j, jnp.int32)
                    wj = jnp.take_along_axis(
                        wg, sel, axis=0, mode="promise_in_bounds")
                    aj = jnp.take_along_axis(
                        sg16, sel, axis=0, mode="promise_in_bounds") | lane_iota
                    plsc.addupdate_scatter(
                        acc_v, [aj], rows2[gp, rbase + j] * wj)

        issue_meta(0, 0)
        wait_meta(0, 0)
        mkidx(0, 0, 0)
        issue_gather(0)
        mkidx(0, 1, 1)
        issue_gather(1)
        issue_meta(1, 1)

        def outer(jj, _):
            for phase in range(2):
                s = 2 * jj + phase
                for k in range(CPS):
                    if k == CPS - 2:
                        wait_meta(s + 1, (phase + 1) % 2)
                    mp2 = phase if k < CPS - 2 else (phase + 1) % 2
                    mkidx(mp2, (k + 2) % CPS, (k + 2) % 4)
                    issue_gather((k + 2) % 4)
                    wait_gather(k)
                    process(phase, k, k)
                    if k == CPS - 1:
                        issue_meta(s + 2, phase)
            return 0

        lax.fori_loop(0, n_super // 2, outer, 0)

        wait_gather(0)
        wait_gather(1)
        wait_meta(n_super + 1, 1)

        pltpu.sync_copy(acc_v, out_hbm.at[cid, sid])

    return sc_kernel(u_tiles, src, dst, w)


TC_BLK = 512


def _tc_finish_body(acc_ref, w_ref, b_ref, out_ref, ssq_ref):
    i = pl.program_id(0)

    @pl.when(i == 0)
    def _():
        ssq_ref[0] = 0.0

    a = acc_ref[0] + acc_ref[1]
    half = NUM_SUBCORES // 2
    left = jnp.concatenate([a[t] for t in range(half)], axis=1)
    right = jnp.concatenate([a[t] for t in range(half, NUM_SUBCORES)], axis=1)
    s = jnp.dot(left, w_ref[...], preferred_element_type=jnp.float32) + b_ref[...]
    t = jnp.dot(right, w_ref[...], preferred_element_type=jnp.float32) + b_ref[...]
    diff = jnp.maximum(s, 0.0) - jnp.maximum(t, 0.0)
    ssq_ref[0] += jnp.sum(diff * diff)

    @pl.when(i == pl.num_programs(0) - 1)
    def _():
        out_ref[...] = jnp.exp(-jnp.sqrt(ssq_ref[0])).reshape(1, 1)


def kernel(x, edge_index, edge_weight, feat_a, W, b):
    n, d_in = x.shape
    d_hid = W.shape[1]
    u = jnp.concatenate([x, feat_a], axis=1)
    u_tiles = u.reshape(n * NUM_SUBCORES, LANES)
    acc = _sc_spmm(u_tiles, edge_index[0], edge_index[1], edge_weight, n)
    acc = acc.reshape(NUM_CORES, NUM_SUBCORES, n, LANES)
    out = pl.pallas_call(
        _tc_finish_body,
        grid=(n // TC_BLK,),
        in_specs=[
            pl.BlockSpec((NUM_CORES, NUM_SUBCORES, TC_BLK, LANES),
                         lambda i: (0, 0, i, 0)),
            pl.BlockSpec((d_in, d_hid), lambda i: (0, 0)),
            pl.BlockSpec((1, d_hid), lambda i: (0, 0)),
        ],
        out_specs=pl.BlockSpec((1, 1), lambda i: (0, 0)),
        out_shape=jax.ShapeDtypeStruct((1, 1), jnp.float32),
        scratch_shapes=[pltpu.SMEM((1,), jnp.float32)],
    )(acc, W, b.reshape(1, -1))
    return out[0, 0]


# speedup vs baseline: 5.1636x; 1.2796x over previous
import functools

import jax
import jax.numpy as jnp
from jax import lax
from jax.experimental import pallas as pl
from jax.experimental.pallas import tpu as pltpu
from jax.experimental.pallas import tpu_sc as plsc

NUM_CORES = 2
NUM_SUBCORES = 16
LANES = 16
CHUNK = 128
SUPER = 512
CPS = SUPER // CHUNK


def _sc_spmm(u_tiles, src, dst, w, n):
    e = w.shape[0]
    e_half = e // NUM_CORES
    n_super = e_half // SUPER

    mesh = plsc.VectorSubcoreMesh(
        core_axis_name="c", subcore_axis_name="s",
        num_cores=NUM_CORES, num_subcores=NUM_SUBCORES)

    @functools.partial(
        pl.kernel,
        out_type=jax.ShapeDtypeStruct(
            (NUM_CORES, NUM_SUBCORES, n * LANES), jnp.float32),
        mesh=mesh,
        compiler_params=pltpu.CompilerParams(
            use_tc_tiling_on_sc=False, needs_layout_passes=False),
        scratch_types=[
            pltpu.VMEM((2, SUPER), jnp.int32),
            pltpu.VMEM((2, SUPER), jnp.int32),
            pltpu.VMEM((2, SUPER), jnp.float32),
            pltpu.VMEM((4, CHUNK), jnp.int32),
            pltpu.VMEM((4, CHUNK, LANES), jnp.float32),
            pltpu.VMEM((n * LANES,), jnp.float32),
            pltpu.SemaphoreType.DMA,
            pltpu.SemaphoreType.DMA,
            pltpu.SemaphoreType.DMA,
            pltpu.SemaphoreType.DMA,
            pltpu.SemaphoreType.DMA,
            pltpu.SemaphoreType.DMA,
        ],
    )
    def sc_kernel(u_hbm, src_hbm, dst_hbm, w_hbm, out_hbm,
                  dst2, src2, w2, idx2, rows2, acc_v,
                  msem0, msem1, gsem0, gsem1, gsem2, gsem3):
        msem = (msem0, msem1)
        gsem = (gsem0, gsem1, gsem2, gsem3)
        cid = lax.axis_index("c")
        sid = lax.axis_index("s")
        ebase = cid * e_half

        zero = jnp.zeros((LANES,), jnp.float32)

        @plsc.parallel_loop(0, n * LANES, step=LANES, unroll=8)
        def zero_row(r):
            acc_v[pl.ds(r, LANES)] = zero

        lane_iota = lax.iota(jnp.int32, LANES)

        def meta_descs(t, mp):
            off = ebase + lax.rem(t, n_super) * SUPER
            sl = pl.ds(off, SUPER)
            return ((dst_hbm.at[sl], dst2.at[mp]),
                    (src_hbm.at[sl], src2.at[mp]),
                    (w_hbm.at[sl], w2.at[mp]))

        def issue_meta(t, mp):
            for s_ref, d_ref in meta_descs(t, mp):
                pltpu.async_copy(s_ref, d_ref, msem[mp])

        def wait_meta(t, mp):
            for s_ref, d_ref in meta_descs(t, mp):
                pltpu.make_async_copy(s_ref, d_ref, msem[mp]).wait()

        def mkidx(mp, koff, gp):
            @plsc.parallel_loop(0, CHUNK, step=LANES, unroll=2)
            def body(i):
                idx2[gp, pl.ds(i, LANES)] = (
                    (dst2[mp, pl.ds(koff * CHUNK + i, LANES)] << 4) + sid)

        def issue_gather(gp):
            pltpu.async_copy(u_hbm.at[idx2.at[gp]], rows2.at[gp], gsem[gp])

        def wait_gather(gp):
            pltpu.make_async_copy(
                u_hbm.at[idx2.at[gp]], rows2.at[gp], gsem[gp]).wait()

        def process(mp, k, gp):
            @plsc.parallel_loop(0, CHUNK // LANES, unroll=4)
            def group(g):
                base = k * CHUNK + g * LANES
                wg = w2[mp, pl.ds(base, LANES)]
                sg16 = src2[mp, pl.ds(base, LANES)] << 4
                rbase = g * LANES
                for j in range(LANES):
                    sel = jnp.full((LANES,), j, jnp.int32)
                    wj = jnp.take_along_axis(
                        wg, sel, axis=0, mode="promise_in_bounds")
                    aj = jnp.take_along_axis(
                        sg16, sel, axis=0, mode="promise_in_bounds") | lane_iota
                    plsc.addupdate_scatter(
                        acc_v, [aj], rows2[gp, rbase + j] * wj)

        issue_meta(0, 0)
        wait_meta(0, 0)
        mkidx(0, 0, 0)
        issue_gather(0)
        mkidx(0, 1, 1)
        issue_gather(1)
        mkidx(0, 2, 2)
        issue_gather(2)
        issue_meta(1, 1)

        def outer(jj, _):
            for phase in range(2):
                s = 2 * jj + phase
                for k in range(CPS):
                    if k == CPS - 3:
                        wait_meta(s + 1, (phase + 1) % 2)
                    mp2 = phase if k < CPS - 3 else (phase + 1) % 2
                    mkidx(mp2, (k + 3) % CPS, (k + 3) % 4)
                    issue_gather((k + 3) % 4)
                    wait_gather(k)
                    process(phase, k, k)
                    if k == CPS - 1:
                        issue_meta(s + 2, phase)
            return 0

        lax.fori_loop(0, n_super // 2, outer, 0)

        wait_gather(0)
        wait_gather(1)
        wait_gather(2)
        wait_meta(n_super + 1, 1)

        pltpu.sync_copy(acc_v, out_hbm.at[cid, sid])

    return sc_kernel(u_tiles, src, dst, w)


TC_BLK = 512


def _tc_finish_body(acc_ref, w_ref, b_ref, out_ref, ssq_ref):
    i = pl.program_id(0)

    @pl.when(i == 0)
    def _():
        ssq_ref[0] = 0.0

    a = acc_ref[0] + acc_ref[1]
    half = NUM_SUBCORES // 2
    left = jnp.concatenate([a[t] for t in range(half)], axis=1)
    right = jnp.concatenate([a[t] for t in range(half, NUM_SUBCORES)], axis=1)
    s = jnp.dot(left, w_ref[...], preferred_element_type=jnp.float32) + b_ref[...]
    t = jnp.dot(right, w_ref[...], preferred_element_type=jnp.float32) + b_ref[...]
    diff = jnp.maximum(s, 0.0) - jnp.maximum(t, 0.0)
    ssq_ref[0] += jnp.sum(diff * diff)

    @pl.when(i == pl.num_programs(0) - 1)
    def _():
        out_ref[...] = jnp.exp(-jnp.sqrt(ssq_ref[0])).reshape(1, 1)


def kernel(x, edge_index, edge_weight, feat_a, W, b):
    n, d_in = x.shape
    d_hid = W.shape[1]
    u = jnp.concatenate([x, feat_a], axis=1)
    u_tiles = u.reshape(n * NUM_SUBCORES, LANES)
    acc = _sc_spmm(u_tiles, edge_index[0], edge_index[1], edge_weight, n)
    acc = acc.reshape(NUM_CORES, NUM_SUBCORES, n, LANES)
    out = pl.pallas_call(
        _tc_finish_body,
        grid=(n // TC_BLK,),
        in_specs=[
            pl.BlockSpec((NUM_CORES, NUM_SUBCORES, TC_BLK, LANES),
                         lambda i: (0, 0, i, 0)),
            pl.BlockSpec((d_in, d_hid), lambda i: (0, 0)),
            pl.BlockSpec((1, d_hid), lambda i: (0, 0)),
        ],
        out_specs=pl.BlockSpec((1, 1), lambda i: (0, 0)),
        out_shape=jax.ShapeDtypeStruct((1, 1), jnp.float32),
        scratch_shapes=[pltpu.SMEM((1,), jnp.float32)],
    )(acc, W, b.reshape(1, -1))
    return out[0, 0]
